# baseline (speedup 1.0000x reference)
import functools

import jax
import jax.numpy as jnp
from jax import lax
from jax.experimental import pallas as pl
from jax.experimental.pallas import tpu as pltpu
from jax.experimental.pallas import tpu_sc as plsc

B = 16384
D = 32
H = 256
NUM_TABLES = 6

try:
    _info = plsc.get_sparse_core_info()
    _NC, _NS = _info.num_cores, _info.num_subcores
except Exception:
    _NC, _NS = 2, 16
_NW = _NC * _NS
_BPW = B // _NW


def _sc_gather_body(*refs):
    idx_hbm = refs[0:6]
    tab_hbm = refs[6:12]
    out_hbm = refs[12:18]
    idx_v = refs[18:24]
    rows_v = refs[24:30]
    sem = refs[30]

    wid = lax.axis_index("s") * _NC + lax.axis_index("c")
    base = wid * _BPW

    for t in range(NUM_TABLES):
        pltpu.sync_copy(idx_hbm[t].at[pl.ds(base, _BPW)], idx_v[t])
    copies = []
    for t in range(NUM_TABLES):
        copies.append(pltpu.async_copy(tab_hbm[t].at[idx_v[t]], rows_v[t], sem))
    for c in copies:
        c.wait()
    for t in range(NUM_TABLES):
        pltpu.sync_copy(rows_v[t], out_hbm[t].at[pl.ds(base, _BPW)])


@functools.lru_cache(maxsize=1)
def _sc_gather():
    return pl.kernel(
        _sc_gather_body,
        mesh=plsc.VectorSubcoreMesh(core_axis_name="c", subcore_axis_name="s"),
        compiler_params=pltpu.CompilerParams(use_tc_tiling_on_sc=False),
        out_type=[jax.ShapeDtypeStruct((B, D), jnp.float32)
                  for _ in range(NUM_TABLES)],
        scratch_types=(
            [pltpu.VMEM((_BPW,), jnp.int32) for _ in range(NUM_TABLES)]
            + [pltpu.VMEM((_BPW, D), jnp.float32) for _ in range(NUM_TABLES)]
            + [pltpu.SemaphoreType.DMA]
        ),
    )


BLK = 2048


def _mlp_body(eu, eg, ea, eo, em, en, w1, b1, w2, b2, w3, b3, out):
    x = jnp.concatenate(
        [eu[...], eg[...], ea[...], eo[...], em[...], en[...]], axis=1)
    h = jnp.dot(x, w1[...], preferred_element_type=jnp.float32) + b1[...]
    h = jnp.maximum(h, 0.0)
    h = jnp.dot(h, w2[...], preferred_element_type=jnp.float32) + b2[...]
    h = jnp.maximum(h, 0.0)
    y = jnp.sum(h * w3[...], axis=1) + b3[0, 0]
    out[...] = y


def _mlp(embs, W1, b1, W2, b2, W3, b3):
    emb_spec = pl.BlockSpec((BLK, D), lambda i: (i, 0))
    full = lambda shape: pl.BlockSpec(shape, lambda i: tuple(0 for _ in shape))
    return pl.pallas_call(
        _mlp_body,
        grid=(B // BLK,),
        in_specs=[emb_spec] * NUM_TABLES + [
            full((NUM_TABLES * D, H)),
            full((H,)),
            full((H, H)),
            full((H,)),
            full((1, H)),
            full((1, 1)),
        ],
        out_specs=pl.BlockSpec((BLK,), lambda i: (i,)),
        out_shape=jax.ShapeDtypeStruct((B,), jnp.float32),
        compiler_params=pltpu.CompilerParams(
            dimension_semantics=("arbitrary",),
        ),
    )(*embs, W1, b1, W2, b2, W3, b3)


def kernel(user_ids, gender_ids, age_ids, occupation_ids, movie_ids, genre_ids,
           user_table, gender_table, age_table, occupation_table, movie_table,
           genre_table, W1, b1, W2, b2, W3, b3):
    ids = [jnp.asarray(i, jnp.int32) for i in
           (user_ids, gender_ids, age_ids, occupation_ids, movie_ids, genre_ids)]
    tables = [user_table, gender_table, age_table, occupation_table,
              movie_table, genre_table]
    embs = _sc_gather()(*ids, *tables)
    return _mlp(embs, W1, b1, W2, b2, W3.reshape(1, H), b3.reshape(1, 1))


# speedup vs baseline: 1.2403x; 1.2403x over previous
import functools

import jax
import jax.numpy as jnp
from jax import lax
from jax.experimental import pallas as pl
from jax.experimental.pallas import tpu as pltpu
from jax.experimental.pallas import tpu_sc as plsc

B = 16384
D = 32
H = 256

try:
    _info = plsc.get_sparse_core_info()
    _NC, _NS = _info.num_cores, _info.num_subcores
except Exception:
    _NC, _NS = 2, 16
_NW = _NC * _NS
_BPW = B // _NW


def _sc_gather_body(uid_hbm, mid_hbm, ut_hbm, mt_hbm, ue_hbm, me_hbm,
                    idx_u, idx_m, rows_u, rows_m, sem):
    wid = lax.axis_index("s") * _NC + lax.axis_index("c")
    base = wid * _BPW

    pltpu.sync_copy(uid_hbm.at[pl.ds(base, _BPW)], idx_u)
    pltpu.sync_copy(mid_hbm.at[pl.ds(base, _BPW)], idx_m)
    cu = pltpu.async_copy(ut_hbm.at[idx_u], rows_u, sem)
    cm = pltpu.async_copy(mt_hbm.at[idx_m], rows_m, sem)
    cu.wait()
    cm.wait()
    pltpu.sync_copy(rows_u, ue_hbm.at[pl.ds(base, _BPW)])
    pltpu.sync_copy(rows_m, me_hbm.at[pl.ds(base, _BPW)])


@functools.lru_cache(maxsize=1)
def _sc_gather():
    return pl.kernel(
        _sc_gather_body,
        mesh=plsc.VectorSubcoreMesh(core_axis_name="c", subcore_axis_name="s"),
        compiler_params=pltpu.CompilerParams(use_tc_tiling_on_sc=False),
        out_type=[jax.ShapeDtypeStruct((B, D), jnp.float32) for _ in range(2)],
        scratch_types=(
            [pltpu.VMEM((_BPW,), jnp.int32) for _ in range(2)]
            + [pltpu.VMEM((_BPW, D), jnp.float32) for _ in range(2)]
            + [pltpu.SemaphoreType.DMA]
        ),
    )


BLK = 2048


def _onehot_embed(ids_1d, table):
    oh = (ids_1d.reshape(BLK, 1) ==
          lax.broadcasted_iota(jnp.int32, (BLK, 32), 1)).astype(jnp.float32)
    return jnp.dot(oh, table, preferred_element_type=jnp.float32)


def _mlp_body(eu, em, gid, aid, oid, nid, gt, at_, ot, nt,
              w1, b1, w2, b2, w3, b3, out):
    ge = _onehot_embed(gid[...], gt[...])
    ae = _onehot_embed(aid[...], at_[...])
    oe = _onehot_embed(oid[...], ot[...])
    ne = _onehot_embed(nid[...], nt[...])
    x = jnp.concatenate([eu[...], ge, ae, oe, em[...], ne], axis=1)
    h = jnp.dot(x, w1[...], preferred_element_type=jnp.float32) + b1[...]
    h = jnp.maximum(h, 0.0)
    h = jnp.dot(h, w2[...], preferred_element_type=jnp.float32) + b2[...]
    h = jnp.maximum(h, 0.0)
    y = jnp.sum(h * w3[...], axis=1) + b3[0, 0]
    out[...] = y


def _mlp(eu, em, gid, aid, oid, nid, gt, at_, ot, nt, W1, b1, W2, b2, W3, b3):
    emb_spec = pl.BlockSpec((BLK, D), lambda i: (i, 0))
    id_spec = pl.BlockSpec((BLK,), lambda i: (i,))
    full = lambda shape: pl.BlockSpec(shape, lambda i: tuple(0 for _ in shape))
    return pl.pallas_call(
        _mlp_body,
        grid=(B // BLK,),
        in_specs=[emb_spec, emb_spec] + [id_spec] * 4 + [full((32, D))] * 4 + [
            full((6 * D, H)),
            full((H,)),
            full((H, H)),
            full((H,)),
            full((1, H)),
            full((1, 1)),
        ],
        out_specs=pl.BlockSpec((BLK,), lambda i: (i,)),
        out_shape=jax.ShapeDtypeStruct((B,), jnp.float32),
        compiler_params=pltpu.CompilerParams(
            dimension_semantics=("arbitrary",),
        ),
    )(eu, em, gid, aid, oid, nid, gt, at_, ot, nt, W1, b1, W2, b2, W3, b3)


def _pad32(t):
    return jnp.pad(t, ((0, 32 - t.shape[0]), (0, 0)))


def kernel(user_ids, gender_ids, age_ids, occupation_ids, movie_ids, genre_ids,
           user_table, gender_table, age_table, occupation_table, movie_table,
           genre_table, W1, b1, W2, b2, W3, b3):
    uid = jnp.asarray(user_ids, jnp.int32)
    mid = jnp.asarray(movie_ids, jnp.int32)
    gid = jnp.asarray(gender_ids, jnp.int32)
    aid = jnp.asarray(age_ids, jnp.int32)
    oid = jnp.asarray(occupation_ids, jnp.int32)
    nid = jnp.asarray(genre_ids, jnp.int32)
    eu, em = _sc_gather()(uid, mid, user_table, movie_table)
    return _mlp(eu, em, gid, aid, oid, nid,
                _pad32(gender_table), _pad32(age_table),
                _pad32(occupation_table), _pad32(genre_table),
                W1, b1, W2, b2, W3.reshape(1, H), b3.reshape(1, 1))


# speedup vs baseline: 1.6089x; 1.2972x over previous
import functools

import jax
import jax.numpy as jnp
from jax import lax
from jax.experimental import pallas as pl
from jax.experimental.pallas import tpu as pltpu

B = 16384
D = 32
H = 256

BLK = 1024
NB = B // BLK
UNROLL = 8


def _onehot_embed(ids_1d, table):
    oh = (ids_1d.reshape(BLK, 1) ==
          lax.broadcasted_iota(jnp.int32, (BLK, 32), 1)).astype(jnp.float32)
    return jnp.dot(oh, table, preferred_element_type=jnp.float32)


def _fused_body(uid_hbm, mid_hbm, ut_hbm, mt_hbm,
                gid, aid, oid, nid, gt, at_, ot, nt,
                w1, b1, w2, b2, w3, b3, out,
                idx_sm, ubuf, mbuf, sem_idx, sem_rows):
    g = pl.program_id(0)

    def stage_idx(blk, buf):
        pltpu.make_async_copy(uid_hbm.at[pl.ds(blk * BLK, BLK)],
                              idx_sm.at[buf, 0], sem_idx).start()
        pltpu.make_async_copy(mid_hbm.at[pl.ds(blk * BLK, BLK)],
                              idx_sm.at[buf, 1], sem_idx).start()
        pltpu.make_async_copy(uid_hbm.at[pl.ds(blk * BLK, BLK)],
                              idx_sm.at[buf, 0], sem_idx).wait()
        pltpu.make_async_copy(mid_hbm.at[pl.ds(blk * BLK, BLK)],
                              idx_sm.at[buf, 1], sem_idx).wait()

    def issue_rows(buf):
        def body(i, _):
            for u in range(UNROLL):
                r = i * UNROLL + u
                iu = idx_sm[buf, 0, r]
                im = idx_sm[buf, 1, r]
                pltpu.make_async_copy(ut_hbm.at[pl.ds(iu, 1), :],
                                      ubuf.at[buf, pl.ds(r, 1), :],
                                      sem_rows).start()
                pltpu.make_async_copy(mt_hbm.at[pl.ds(im, 1), :],
                                      mbuf.at[buf, pl.ds(r, 1), :],
                                      sem_rows).start()
            return 0
        lax.fori_loop(0, BLK // UNROLL, body, 0, unroll=True)

    def drain_rows(buf):
        pltpu.make_async_copy(ut_hbm.at[pl.ds(0, BLK), :], ubuf.at[buf],
                              sem_rows).wait()
        pltpu.make_async_copy(mt_hbm.at[pl.ds(0, BLK), :], mbuf.at[buf],
                              sem_rows).wait()

    @pl.when(g == 0)
    def _prologue():
        stage_idx(0, 0)
        issue_rows(0)

    @pl.when(g + 1 < NB)
    def _next_block():
        stage_idx(g + 1, (g + 1) % 2)
        issue_rows((g + 1) % 2)

    drain_rows(g % 2)

    cur = g % 2
    eu = ubuf[cur]
    em = mbuf[cur]
    ge = _onehot_embed(gid[...], gt[...])
    ae = _onehot_embed(aid[...], at_[...])
    oe = _onehot_embed(oid[...], ot[...])
    ne = _onehot_embed(nid[...], nt[...])
    x = jnp.concatenate([eu, ge, ae, oe, em, ne], axis=1)
    h = jnp.dot(x, w1[...], preferred_element_type=jnp.float32) + b1[...]
    h = jnp.maximum(h, 0.0)
    h = jnp.dot(h, w2[...], preferred_element_type=jnp.float32) + b2[...]
    h = jnp.maximum(h, 0.0)
    y = jnp.sum(h * w3[...], axis=1) + b3[0, 0]
    out[...] = y


def _fused(uid, mid, ut, mt, gid, aid, oid, nid, gt, at_, ot, nt,
           W1, b1, W2, b2, W3, b3):
    id_spec = pl.BlockSpec((BLK,), lambda i: (i,))
    any_spec = pl.BlockSpec(memory_space=pl.ANY)
    full = lambda shape: pl.BlockSpec(shape, lambda i: tuple(0 for _ in shape))
    return pl.pallas_call(
        _fused_body,
        grid=(NB,),
        in_specs=[any_spec] * 4 + [id_spec] * 4 + [full((32, D))] * 4 + [
            full((6 * D, H)),
            full((H,)),
            full((H, H)),
            full((H,)),
            full((1, H)),
            full((1, 1)),
        ],
        out_specs=pl.BlockSpec((BLK,), lambda i: (i,)),
        out_shape=jax.ShapeDtypeStruct((B,), jnp.float32),
        scratch_shapes=[
            pltpu.SMEM((2, 2, BLK), jnp.int32),
            pltpu.VMEM((2, BLK, D), jnp.float32),
            pltpu.VMEM((2, BLK, D), jnp.float32),
            pltpu.SemaphoreType.DMA,
            pltpu.SemaphoreType.DMA,
        ],
        compiler_params=pltpu.CompilerParams(
            dimension_semantics=("arbitrary",),
        ),
    )(uid, mid, ut, mt, gid, aid, oid, nid, gt, at_, ot, nt,
      W1, b1, W2, b2, W3, b3)


def _pad32(t):
    return jnp.pad(t, ((0, 32 - t.shape[0]), (0, 0)))


def kernel(user_ids, gender_ids, age_ids, occupation_ids, movie_ids, genre_ids,
           user_table, gender_table, age_table, occupation_table, movie_table,
           genre_table, W1, b1, W2, b2, W3, b3):
    uid = jnp.asarray(user_ids, jnp.int32)
    mid = jnp.asarray(movie_ids, jnp.int32)
    gid = jnp.asarray(gender_ids, jnp.int32)
    aid = jnp.asarray(age_ids, jnp.int32)
    oid = jnp.asarray(occupation_ids, jnp.int32)
    nid = jnp.asarray(genre_ids, jnp.int32)
    return _fused(uid, mid, user_table, movie_table, gid, aid, oid, nid,
                  _pad32(gender_table), _pad32(age_table),
                  _pad32(occupation_table), _pad32(genre_table),
                  W1, b1, W2, b2, W3.reshape(1, H), b3.reshape(1, 1))


# speedup vs baseline: 1.8352x; 1.1407x over previous
import functools

import jax
import jax.numpy as jnp
from jax import lax
from jax.experimental import pallas as pl
from jax.experimental.pallas import tpu as pltpu
from jax.experimental.pallas import tpu_sc as plsc

B = 16384
D = 32
H = 256

try:
    _info = plsc.get_sparse_core_info()
    _NC, _NS = _info.num_cores, _info.num_subcores
except Exception:
    _NC, _NS = 2, 16
_NW = _NC * _NS
_BPW = B // _NW


_CH = 128
_NCHUNK = _BPW // _CH


def _sc_gather_body(uid_hbm, mid_hbm, ut_hbm, mt_hbm, ue_hbm, me_hbm,
                    idx_vm, rows_u, rows_m, sem, osem):
    wid = lax.axis_index("s") * _NC + lax.axis_index("c")
    base = wid * _BPW

    pltpu.sync_copy(uid_hbm.at[pl.ds(base, _BPW)], idx_vm.at[0])
    pltpu.sync_copy(mid_hbm.at[pl.ds(base, _BPW)], idx_vm.at[1])

    def fire(c, buf):
        def step(k, _):
            vu = idx_vm[0, pl.ds(c * _CH + k * 16, 16)]
            vm = idx_vm[1, pl.ds(c * _CH + k * 16, 16)]
            for lane in range(16):
                iu = jnp.squeeze(lax.slice(vu, (lane,), (lane + 1,)))
                im = jnp.squeeze(lax.slice(vm, (lane,), (lane + 1,)))
                pltpu.async_copy(ut_hbm.at[pl.ds(iu, 1), :],
                                 rows_u.at[buf, pl.ds(k * 16 + lane, 1), :],
                                 sem)
                pltpu.async_copy(mt_hbm.at[pl.ds(im, 1), :],
                                 rows_m.at[buf, pl.ds(k * 16 + lane, 1), :],
                                 sem)
            return 0
        lax.fori_loop(0, _CH // 16, step, 0)

    def drain(buf):
        pltpu.make_async_copy(ut_hbm.at[pl.ds(0, _CH), :],
                              rows_u.at[buf], sem).wait()
        pltpu.make_async_copy(mt_hbm.at[pl.ds(0, _CH), :],
                              rows_m.at[buf], sem).wait()

    for c in range(_NCHUNK):
        fire(c, 0)
        drain(0)
        pltpu.sync_copy(rows_u.at[0],
                        ue_hbm.at[pl.ds(base + c * _CH, _CH)])
        pltpu.sync_copy(rows_m.at[0],
                        me_hbm.at[pl.ds(base + c * _CH, _CH)])


@functools.lru_cache(maxsize=1)
def _sc_gather():
    return pl.kernel(
        _sc_gather_body,
        mesh=plsc.VectorSubcoreMesh(core_axis_name="c", subcore_axis_name="s"),
        out_type=[jax.ShapeDtypeStruct((B, D), jnp.float32) for _ in range(2)],
        scratch_types=[
            pltpu.VMEM((2, _BPW), jnp.int32),
            pltpu.VMEM((2, _CH, D), jnp.float32),
            pltpu.VMEM((2, _CH, D), jnp.float32),
            pltpu.SemaphoreType.DMA,
            pltpu.SemaphoreType.DMA,
        ],
    )


BLK = 2048


def _onehot_embed(ids_1d, table):
    oh = (ids_1d.reshape(BLK, 1) ==
          lax.broadcasted_iota(jnp.int32, (BLK, 32), 1)).astype(jnp.float32)
    return jnp.dot(oh, table, preferred_element_type=jnp.float32)


def _mlp_body(eu, em, gid, aid, oid, nid, gt, at_, ot, nt,
              w1, b1, w2, b2, w3, b3, out):
    ge = _onehot_embed(gid[...], gt[...])
    ae = _onehot_embed(aid[...], at_[...])
    oe = _onehot_embed(oid[...], ot[...])
    ne = _onehot_embed(nid[...], nt[...])
    x = jnp.concatenate([eu[...], ge, ae, oe, em[...], ne], axis=1)
    h = jnp.dot(x, w1[...], preferred_element_type=jnp.float32) + b1[...]
    h = jnp.maximum(h, 0.0)
    h = jnp.dot(h, w2[...], preferred_element_type=jnp.float32) + b2[...]
    h = jnp.maximum(h, 0.0)
    y = jnp.sum(h * w3[...], axis=1) + b3[0, 0]
    out[...] = y


def _mlp(eu, em, gid, aid, oid, nid, gt, at_, ot, nt, W1, b1, W2, b2, W3, b3):
    emb_spec = pl.BlockSpec((BLK, D), lambda i: (i, 0))
    id_spec = pl.BlockSpec((BLK,), lambda i: (i,))
    full = lambda shape: pl.BlockSpec(shape, lambda i: tuple(0 for _ in shape))
    return pl.pallas_call(
        _mlp_body,
        grid=(B // BLK,),
        in_specs=[emb_spec, emb_spec] + [id_spec] * 4 + [full((32, D))] * 4 + [
            full((6 * D, H)),
            full((H,)),
            full((H, H)),
            full((H,)),
            full((1, H)),
            full((1, 1)),
        ],
        out_specs=pl.BlockSpec((BLK,), lambda i: (i,)),
        out_shape=jax.ShapeDtypeStruct((B,), jnp.float32),
        compiler_params=pltpu.CompilerParams(
            dimension_semantics=("arbitrary",),
        ),
    )(eu, em, gid, aid, oid, nid, gt, at_, ot, nt, W1, b1, W2, b2, W3, b3)


def _pad32(t):
    return jnp.pad(t, ((0, 32 - t.shape[0]), (0, 0)))


def kernel(user_ids, gender_ids, age_ids, occupation_ids, movie_ids, genre_ids,
           user_table, gender_table, age_table, occupation_table, movie_table,
           genre_table, W1, b1, W2, b2, W3, b3):
    uid = jnp.asarray(user_ids, jnp.int32)
    mid = jnp.asarray(movie_ids, jnp.int32)
    gid = jnp.asarray(gender_ids, jnp.int32)
    aid = jnp.asarray(age_ids, jnp.int32)
    oid = jnp.asarray(occupation_ids, jnp.int32)
    nid = jnp.asarray(genre_ids, jnp.int32)
    eu, em = _sc_gather()(uid, mid, user_table, movie_table)
    return _mlp(eu, em, gid, aid, oid, nid,
                _pad32(gender_table), _pad32(age_table),
                _pad32(occupation_table), _pad32(genre_table),
                W1, b1, W2, b2, W3.reshape(1, H), b3.reshape(1, 1))


# speedup vs baseline: 2.1673x; 1.1810x over previous
import functools

import jax
import jax.numpy as jnp
from jax import lax
from jax.experimental import pallas as pl
from jax.experimental.pallas import tpu as pltpu
from jax.experimental.pallas import tpu_sc as plsc

B = 16384
D = 32
H = 256

try:
    _info = plsc.get_sparse_core_info()
    _NC, _NS = _info.num_cores, _info.num_subcores
except Exception:
    _NC, _NS = 2, 16
_NW = _NC * _NS
_BPW = B // _NW


_CH = 128
_NCHUNK = _BPW // _CH


def _sc_gather_body(uid_hbm, mid_hbm, utT_hbm, mtT_hbm, ueT_hbm, meT_hbm,
                    idx_vm, cols_u, cols_m, sem, osem):
    wid = lax.axis_index("s") * _NC + lax.axis_index("c")
    base = wid * _BPW

    pltpu.sync_copy(uid_hbm.at[pl.ds(base, _BPW)], idx_vm.at[0])
    pltpu.sync_copy(mid_hbm.at[pl.ds(base, _BPW)], idx_vm.at[1])

    def fire(c, buf):
        def step(k, _):
            vu = idx_vm[0, pl.ds(c * _CH + k * 16, 16)]
            vm = idx_vm[1, pl.ds(c * _CH + k * 16, 16)]
            for lane in range(16):
                iu = jnp.squeeze(lax.slice(vu, (lane,), (lane + 1,)))
                im = jnp.squeeze(lax.slice(vm, (lane,), (lane + 1,)))
                pltpu.async_copy(utT_hbm.at[pl.ds(iu, 1), :],
                                 cols_u.at[buf, pl.ds(k * 16 + lane, 1), :],
                                 sem)
                pltpu.async_copy(mtT_hbm.at[pl.ds(im, 1), :],
                                 cols_m.at[buf, pl.ds(k * 16 + lane, 1), :],
                                 sem)
            return 0
        lax.fori_loop(0, _CH // 16, step, 0)

    def drain(buf):
        pltpu.make_async_copy(utT_hbm.at[pl.ds(0, _CH), :],
                              cols_u.at[buf], sem).wait()
        pltpu.make_async_copy(mtT_hbm.at[pl.ds(0, _CH), :],
                              cols_m.at[buf], sem).wait()

    for c in range(_NCHUNK):
        fire(c, 0)
        drain(0)
        pltpu.sync_copy(cols_u.at[0],
                        ueT_hbm.at[pl.ds(base + c * _CH, _CH)])
        pltpu.sync_copy(cols_m.at[0],
                        meT_hbm.at[pl.ds(base + c * _CH, _CH)])


@functools.lru_cache(maxsize=1)
def _sc_gather():
    return pl.kernel(
        _sc_gather_body,
        mesh=plsc.VectorSubcoreMesh(core_axis_name="c", subcore_axis_name="s"),
        out_type=[jax.ShapeDtypeStruct((B, D), jnp.float32) for _ in range(2)],
        scratch_types=[
            pltpu.VMEM((2, _BPW), jnp.int32),
            pltpu.VMEM((2, _CH, D), jnp.float32),
            pltpu.VMEM((2, _CH, D), jnp.float32),
            pltpu.SemaphoreType.DMA,
            pltpu.SemaphoreType.DMA,
        ],
    )


_TC = 8192
_NROWS = 1000000


def _transpose_body(src, dst):
    dst[...] = jnp.transpose(src[...], (1, 0))


def _transpose_table(tT):
    n = tT.shape[1]
    grid = (n + _TC - 1) // _TC
    return pl.pallas_call(
        _transpose_body,
        grid=(grid,),
        in_specs=[pl.BlockSpec((D, _TC), lambda i: (0, i))],
        out_specs=pl.BlockSpec((_TC, D), lambda i: (i, 0)),
        out_shape=jax.ShapeDtypeStruct((n, D), jnp.float32),
        compiler_params=pltpu.CompilerParams(
            dimension_semantics=("arbitrary",),
        ),
    )(tT)


BLK = 2048


def _onehot_embed(ids_1d, table):
    oh = (ids_1d.reshape(BLK, 1) ==
          lax.broadcasted_iota(jnp.int32, (BLK, 32), 1)).astype(jnp.float32)
    return jnp.dot(oh, table, preferred_element_type=jnp.float32)


def _mlp_body(eu, em, gid, aid, oid, nid, gt, at_, ot, nt,
              w1, b1, w2, b2, w3, b3, out):
    ge = _onehot_embed(gid[...], gt[...])
    ae = _onehot_embed(aid[...], at_[...])
    oe = _onehot_embed(oid[...], ot[...])
    ne = _onehot_embed(nid[...], nt[...])
    x = jnp.concatenate([eu[...], ge, ae, oe, em[...], ne], axis=1)
    h = jnp.dot(x, w1[...], preferred_element_type=jnp.float32) + b1[...]
    h = jnp.maximum(h, 0.0)
    h = jnp.dot(h, w2[...], preferred_element_type=jnp.float32) + b2[...]
    h = jnp.maximum(h, 0.0)
    y = jnp.sum(h * w3[...], axis=1) + b3[0, 0]
    out[...] = y


def _mlp(eu, em, gid, aid, oid, nid, gt, at_, ot, nt, W1, b1, W2, b2, W3, b3):
    emb_spec = pl.BlockSpec((BLK, D), lambda i: (i, 0))
    id_spec = pl.BlockSpec((BLK,), lambda i: (i,))
    full = lambda shape: pl.BlockSpec(shape, lambda i: tuple(0 for _ in shape))
    return pl.pallas_call(
        _mlp_body,
        grid=(B // BLK,),
        in_specs=[emb_spec, emb_spec] + [id_spec] * 4 + [full((32, D))] * 4 + [
            full((6 * D, H)),
            full((H,)),
            full((H, H)),
            full((H,)),
            full((1, H)),
            full((1, 1)),
        ],
        out_specs=pl.BlockSpec((BLK,), lambda i: (i,)),
        out_shape=jax.ShapeDtypeStruct((B,), jnp.float32),
        compiler_params=pltpu.CompilerParams(
            dimension_semantics=("arbitrary",),
        ),
    )(eu, em, gid, aid, oid, nid, gt, at_, ot, nt, W1, b1, W2, b2, W3, b3)


def _pad32(t):
    return jnp.pad(t, ((0, 32 - t.shape[0]), (0, 0)))


def kernel(user_ids, gender_ids, age_ids, occupation_ids, movie_ids, genre_ids,
           user_table, gender_table, age_table, occupation_table, movie_table,
           genre_table, W1, b1, W2, b2, W3, b3):
    uid = jnp.asarray(user_ids, jnp.int32)
    mid = jnp.asarray(movie_ids, jnp.int32)
    gid = jnp.asarray(gender_ids, jnp.int32)
    aid = jnp.asarray(age_ids, jnp.int32)
    oid = jnp.asarray(occupation_ids, jnp.int32)
    nid = jnp.asarray(genre_ids, jnp.int32)
    ut_rm = _transpose_table(user_table.T)
    mt_rm = _transpose_table(movie_table.T)
    eu, em = _sc_gather()(uid, mid, ut_rm, mt_rm)
    return _mlp(eu, em, gid, aid, oid, nid,
                _pad32(gender_table), _pad32(age_table),
                _pad32(occupation_table), _pad32(genre_table),
                W1, b1, W2, b2, W3.reshape(1, H), b3.reshape(1, 1))
